# baseline (speedup 1.0000x reference)
import jax
import jax.numpy as jnp
from jax.experimental import pallas as pl
from jax.experimental.pallas import tpu as pltpu
from jax.experimental.pallas import tpu_sc as plsc

F32 = jnp.float32
BF16 = jnp.bfloat16

_B, _T, _D, _DFF, _H, _E, _K = 2, 2048, 768, 3072, 12, 8, 2
_DH = _D // _H
_N = _B * _T
_NP = _N * _K
_BM = 512
_NBLK = _NP // _BM + _E
_NPAD = _NBLK * _BM
_BQ = 1024
_DH2 = _D // 2


def _qkv_body(x_ref, w_ref, b_ref, o_ref):
    acc = jax.lax.dot_general(x_ref[...].astype(BF16), w_ref[...],
                              (((1,), (0,)), ((), ())),
                              preferred_element_type=F32)
    o_ref[...] = (acc + b_ref[...]).astype(BF16)


def _qkv_proj(xb, wqkv, bqkv):
    bm = 512
    return pl.pallas_call(
        _qkv_body,
        grid=(_N // bm,),
        in_specs=[pl.BlockSpec((bm, _D), lambda i: (i, 0)),
                  pl.BlockSpec((_D, 3 * _D), lambda i: (0, 0)),
                  pl.BlockSpec((1, 3 * _D), lambda i: (0, 0))],
        out_specs=pl.BlockSpec((bm, 3 * _D), lambda i: (i, 0)),
        out_shape=jax.ShapeDtypeStruct((_N, 3 * _D), BF16),
    )(xb, wqkv, bqkv)


def _attn_body(q_ref, k_ref, v_ref, a_ref, c_ref):
    qq = q_ref[...]
    kk = k_ref[...]
    vv = v_ref[...]
    outs = []
    for hh in range(2):
        sl = slice(hh * _DH, (hh + 1) * _DH)
        s = jax.lax.dot_general(qq[:, sl], kk[:, sl], (((1,), (1,)), ((), ())),
                                preferred_element_type=F32)
        p = jnp.exp2(s * 0.18033688011112042)
        p = p * (1.0 / jnp.sum(p, axis=-1, keepdims=True))
        a_ref[0, hh] = p
        outs.append(jax.lax.dot_general(p.astype(BF16), vv[:, sl],
                                        (((1,), (0,)), ((), ())),
                                        preferred_element_type=F32))
    c_ref[...] = jnp.concatenate(outs, axis=1).astype(BF16)


def _attention(qkv):
    nj = _T // _BQ
    hp = _H // 2
    return pl.pallas_call(
        _attn_body,
        grid=(_B, hp, nj),
        in_specs=[
            pl.BlockSpec((_BQ, 2 * _DH), lambda b, h, j, _nj=nj: (b * _nj + j, h)),
            pl.BlockSpec((_T, 2 * _DH), lambda b, h, j, _hp=hp: (b, _hp + h)),
            pl.BlockSpec((_T, 2 * _DH), lambda b, h, j, _hp=hp: (b, 2 * _hp + h)),
        ],
        out_specs=[
            pl.BlockSpec((1, 2, _BQ, _T), lambda b, h, j: (b, h, j, 0)),
            pl.BlockSpec((_BQ, 2 * _DH), lambda b, h, j, _nj=nj: (b * _nj + j, h)),
        ],
        out_shape=[
            jax.ShapeDtypeStruct((_B, _H, _T, _T), F32),
            jax.ShapeDtypeStruct((_N, _D), BF16),
        ],
    )(qkv, qkv, qkv)


def _post_body(ctx_ref, x_ref, wo_ref, bo_ref, g1_ref, be1_ref, wg_ref,
               x12_ref, gate_ref):
    nx = jax.lax.dot_general(ctx_ref[...], wo_ref[...], (((1,), (0,)), ((), ())),
                             preferred_element_type=F32) + bo_ref[...]
    x1 = x_ref[...] + nx
    mu = jnp.mean(x1, axis=-1, keepdims=True)
    xc = x1 - mu
    var = jnp.mean(xc * xc, axis=-1, keepdims=True)
    x1 = xc / jnp.sqrt(var + 1e-5) * g1_ref[...] + be1_ref[...]
    x12_ref[0, 0] = x1[:, :_DH2]
    x12_ref[1, 0] = x1[:, _DH2:]
    logits = jax.lax.dot_general(x1, wg_ref[...], (((1,), (0,)), ((), ())),
                                 preferred_element_type=F32)
    mask = jax.lax.broadcasted_iota(jnp.int32, logits.shape, 1) < _E
    logits = jnp.where(mask, logits, -1e30)
    mx = jnp.max(logits, axis=-1, keepdims=True)
    p = jnp.exp(logits - mx)
    gate_ref[...] = p / jnp.sum(p, axis=-1, keepdims=True)


def _post_attn(ctx2, xf, wob, bo2, g12, be12, wgp):
    bm = 512
    return pl.pallas_call(
        _post_body,
        grid=(_N // bm,),
        in_specs=[
            pl.BlockSpec((bm, _D), lambda i: (i, 0)),
            pl.BlockSpec((bm, _D), lambda i: (i, 0)),
            pl.BlockSpec((_D, _D), lambda i: (0, 0)),
            pl.BlockSpec((1, _D), lambda i: (0, 0)),
            pl.BlockSpec((1, _D), lambda i: (0, 0)),
            pl.BlockSpec((1, _D), lambda i: (0, 0)),
            pl.BlockSpec((_D, 128), lambda i: (0, 0)),
        ],
        out_specs=[
            pl.BlockSpec((2, 1, bm, _DH2), lambda i: (0, i, 0, 0)),
            pl.BlockSpec((bm, 128), lambda i: (i, 0)),
        ],
        out_shape=[
            jax.ShapeDtypeStruct((2, _N // bm, bm, _DH2), F32),
            jax.ShapeDtypeStruct((_N, 128), F32),
        ],
    )(ctx2, xf, wob, bo2, g12, be12, wgp)


def _sc_gather(table, idx, n_rows, window=128):
    mesh = plsc.VectorSubcoreMesh(core_axis_name="core",
                                  subcore_axis_name="subcore")
    width = table.shape[1]
    idx2 = idx.reshape(1, n_rows)

    @pl.kernel(out_type=jax.ShapeDtypeStruct((n_rows, width), table.dtype),
               mesh=mesh)
    def k(x_hbm, i_hbm, o_hbm):
        def body(i_vmem, o_vmem):
            pltpu.sync_copy(x_hbm.at[i_vmem.at[0]], o_vmem)

        pltpu.emit_pipeline(
            body,
            grid=(n_rows // window,),
            in_specs=[pl.BlockSpec((1, window), lambda i: (0, i))],
            out_specs=[pl.BlockSpec((window, width), lambda i: (i, 0))],
            core_axis_name=("core", "subcore"),
            dimension_semantics=(pltpu.PARALLEL,),
        )(i_hbm, o_hbm)

    return k(table, idx2)


def _sc_scatter(x2, ia, ib, n_slots, window=128):
    mesh = plsc.VectorSubcoreMesh(core_axis_name="core",
                                  subcore_axis_name="subcore")
    n_rows, width = x2.shape
    ia2 = ia.reshape(1, n_rows)
    ib2 = ib.reshape(1, n_rows)

    @pl.kernel(out_type=jax.ShapeDtypeStruct((n_slots, width), x2.dtype),
               mesh=mesh)
    def k(x_hbm, ia_hbm, ib_hbm, o_hbm):
        def body(x_vmem, ia_vmem, ib_vmem):
            pltpu.sync_copy(x_vmem, o_hbm.at[ia_vmem.at[0]])
            pltpu.sync_copy(x_vmem, o_hbm.at[ib_vmem.at[0]])

        pltpu.emit_pipeline(
            body,
            grid=(n_rows // window,),
            in_specs=[
                pl.BlockSpec((window, width), lambda i: (i, 0)),
                pl.BlockSpec((1, window), lambda i: (0, i)),
                pl.BlockSpec((1, window), lambda i: (0, i)),
            ],
            out_specs=[],
            core_axis_name=("core", "subcore"),
            dimension_semantics=(pltpu.PARALLEL,),
        )(x_hbm, ia_hbm, ib_hbm)

    return k(x2, ia2, ib2)


def _ffn_body(be_ref, bn_ref, lo_ref, hi_ref, w1lo_ref, w1hi_ref,
              b1_ref, w2_ref, b2_ref, ov_ref, w1lo_bf, w1hi_bf, w2_bf):
    i = pl.program_id(0)
    prev = jnp.maximum(i - 1, 0)
    changed = (i == 0) | (be_ref[i] != be_ref[prev])

    @pl.when(changed)
    def _():
        w1lo_bf[...] = w1lo_ref[0, 0].astype(BF16)
        w1hi_bf[...] = w1hi_ref[0, 0].astype(BF16)
        w2_bf[...] = w2_ref[0].astype(BF16)

    def run_rows(base, nrows):
        sl = pl.ds(base, nrows)
        h = jax.lax.dot_general(lo_ref[sl, :].astype(BF16), w1lo_bf[...],
                                (((1,), (0,)), ((), ())),
                                preferred_element_type=F32)
        h += jax.lax.dot_general(hi_ref[sl, :].astype(BF16), w1hi_bf[...],
                                 (((1,), (0,)), ((), ())),
                                 preferred_element_type=F32)
        h = jnp.maximum(h + b1_ref[0], 0.0).astype(BF16)
        o = jax.lax.dot_general(h, w2_bf[...], (((1,), (0,)), ((), ())),
                                preferred_element_type=F32)
        o = o + b2_ref[0]
        ov_ref[0, 0, sl, :] = o[:, :_DH2]
        ov_ref[1, 0, sl, :] = o[:, _DH2:]

    @pl.when(bn_ref[i] > 0)
    def _():
        run_rows(0, _BM // 2)

    @pl.when(bn_ref[i] > _BM // 2)
    def _():
        run_rows(_BM // 2, _BM // 2)


def _ffn(block_expert, block_nrows, rows2, w1b, b1r, w2b, b2r):
    grid_spec = pltpu.PrefetchScalarGridSpec(
        num_scalar_prefetch=2,
        grid=(_NBLK,),
        in_specs=[
            pl.BlockSpec((_BM, _DH2), lambda i, be, bn: (i, 0)),
            pl.BlockSpec((_BM, _DH2), lambda i, be, bn: (i + _NBLK, 0)),
            pl.BlockSpec((1, 1, _DH2, _DFF), lambda i, be, bn: (be[i], 0, 0, 0)),
            pl.BlockSpec((1, 1, _DH2, _DFF), lambda i, be, bn: (be[i], 1, 0, 0)),
            pl.BlockSpec((1, 1, _DFF), lambda i, be, bn: (be[i], 0, 0)),
            pl.BlockSpec((1, _DFF, _D), lambda i, be, bn: (be[i], 0, 0)),
            pl.BlockSpec((1, 1, _D), lambda i, be, bn: (be[i], 0, 0)),
        ],
        out_specs=pl.BlockSpec((2, 1, _BM, _DH2), lambda i, be, bn: (0, i, 0, 0)),
        scratch_shapes=[
            pltpu.VMEM((_DH2, _DFF), BF16),
            pltpu.VMEM((_DH2, _DFF), BF16),
            pltpu.VMEM((_DFF, _D), BF16),
        ],
    )
    return pl.pallas_call(
        _ffn_body,
        grid_spec=grid_spec,
        out_shape=jax.ShapeDtypeStruct((2, _NBLK, _BM, _DH2), F32),
    )(block_expert, block_nrows, rows2, rows2, w1b, w1b, b1r, w2b, b2r)


def _final_body(xlo_ref, xhi_ref, galo_ref, gahi_ref, gblo_ref, gbhi_ref,
                wa_ref, wb_ref, g2_ref, be2_ref, o_ref):
    wa = wa_ref[:, 0:1]
    wb = wb_ref[:, 0:1]
    slo = xlo_ref[...] + wa * galo_ref[...] + wb * gblo_ref[...]
    shi = xhi_ref[...] + wa * gahi_ref[...] + wb * gbhi_ref[...]
    mu = (jnp.sum(slo, axis=-1, keepdims=True)
          + jnp.sum(shi, axis=-1, keepdims=True)) * (1.0 / _D)
    clo = slo - mu
    chi = shi - mu
    var = (jnp.sum(clo * clo, axis=-1, keepdims=True)
           + jnp.sum(chi * chi, axis=-1, keepdims=True)) * (1.0 / _D)
    r = 1.0 / jnp.sqrt(var + 1e-5)
    o_ref[:, :_DH2] = clo * r * g2_ref[:, :_DH2] + be2_ref[:, :_DH2]
    o_ref[:, _DH2:] = chi * r * g2_ref[:, _DH2:] + be2_ref[:, _DH2:]


def _final(x12f, g4, wab, g22, be22):
    bm = 512
    nb = _N // bm
    return pl.pallas_call(
        _final_body,
        grid=(nb,),
        in_specs=[
            pl.BlockSpec((bm, _DH2), lambda i: (i, 0)),
            pl.BlockSpec((bm, _DH2), lambda i, _nb=nb: (i + _nb, 0)),
            pl.BlockSpec((bm, _DH2), lambda i: (i, 0)),
            pl.BlockSpec((bm, _DH2), lambda i, _nb=nb: (i + 2 * _nb, 0)),
            pl.BlockSpec((bm, _DH2), lambda i, _nb=nb: (i + _nb, 0)),
            pl.BlockSpec((bm, _DH2), lambda i, _nb=nb: (i + 3 * _nb, 0)),
            pl.BlockSpec((bm, 128), lambda i: (i, 0)),
            pl.BlockSpec((bm, 128), lambda i, _nb=nb: (i + _nb, 0)),
            pl.BlockSpec((1, _D), lambda i: (0, 0)),
            pl.BlockSpec((1, _D), lambda i: (0, 0)),
        ],
        out_specs=pl.BlockSpec((bm, _D), lambda i: (i, 0)),
        out_shape=jax.ShapeDtypeStruct((_N, _D), F32),
    )(x12f, x12f, g4, g4, g4, g4, wab, wab, g22, be22)


def _routing(probs):
    w, sel = jax.lax.top_k(probs, _K)
    e_flat = sel.reshape(_NP).astype(jnp.int32)
    oh = (e_flat[:, None] == jnp.arange(_E, dtype=jnp.int32)[None, :])
    oh = oh.astype(F32)
    ch = 128
    nch = _NP // ch
    oh3 = oh.reshape(nch, ch, _E)
    ii = jax.lax.broadcasted_iota(jnp.int32, (ch, ch), 0)
    jj = jax.lax.broadcasted_iota(jnp.int32, (ch, ch), 1)
    tril = (ii >= jj).astype(F32)
    within = jnp.einsum("ij,bjk->bik", tril, oh3,
                        preferred_element_type=F32)
    chunk_tot = jnp.sum(oh3, axis=1)
    i2 = jax.lax.broadcasted_iota(jnp.int32, (nch, nch), 0)
    j2 = jax.lax.broadcasted_iota(jnp.int32, (nch, nch), 1)
    stril = (i2 > j2).astype(F32)
    offs = jnp.einsum("ij,jk->ik", stril, chunk_tot,
                      preferred_element_type=F32)
    ranks = (within + offs[:, None, :]).reshape(_NP, _E)
    c = (jnp.sum(chunk_tot, axis=0)).astype(jnp.int32)
    blocks = (c + _BM - 1) // _BM
    cum_blocks = jnp.cumsum(blocks)
    pad_start = _BM * (cum_blocks - blocks)
    rank_p = jnp.sum(ranks * oh, axis=1).astype(jnp.int32) - 1
    dest = pad_start[e_flat] + rank_p
    bidx = jnp.arange(_NBLK, dtype=jnp.int32)
    block_expert = jnp.minimum(
        jnp.searchsorted(cum_blocks, bidx, side="right"), _E - 1
    ).astype(jnp.int32)
    off = bidx * _BM - pad_start[block_expert]
    block_nrows = jnp.clip(c[block_expert] - off, 0, _BM).astype(jnp.int32)
    return w, dest, block_expert, block_nrows


def kernel(x, Wq, bq, Wk, bk, Wv, bv, Wo, bo, g1, be1, g2, be2, Wg, W1, b1,
           W2, b2):
    xf = x.reshape(_N, _D)
    wqkv = jnp.concatenate([Wq, Wk, Wv], axis=1).astype(BF16)
    bqkv = jnp.concatenate([bq, bk, bv])[None, :]
    qkv = _qkv_proj(xf, wqkv, bqkv)
    attn, ctx2 = _attention(qkv)

    wgp = jnp.pad(Wg, ((0, 0), (0, 128 - _E)))
    x12, gate = _post_attn(ctx2, xf, Wo.astype(BF16), bo[None, :],
                           g1[None, :], be1[None, :], wgp)
    x12f = x12.reshape(2 * _N, _DH2)
    probs = gate[:, :_E]

    w, dest, block_expert, block_nrows = _routing(probs)

    d2 = dest.reshape(_N, _K)
    ia = jnp.concatenate([d2[:, 0], d2[:, 0] + _NPAD])
    ib = jnp.concatenate([d2[:, 1], d2[:, 1] + _NPAD])
    rows2 = _sc_scatter(x12f, ia, ib, 2 * _NPAD)
    ov = _ffn(block_expert, block_nrows, rows2,
              W1.reshape(_E, 2, _DH2, _DFF), b1.reshape(_E, 1, _DFF),
              W2, b2.reshape(_E, 1, _D))
    ovf = ov.reshape(2 * _NPAD, _DH2)

    idx2 = d2.T.reshape(_NP)
    gidx = jnp.concatenate([idx2, idx2 + _NPAD])
    g4 = _sc_gather(ovf, gidx, 2 * _NP)

    wab = jnp.broadcast_to(w.T.reshape(_NP, 1), (_NP, 128))
    out = _final(x12f, g4, wab, g2[None, :], be2[None, :])
    return (out.reshape(_B, _T, _D), attn, probs.reshape(_B, _T, _E))


# speedup vs baseline: 1.0104x; 1.0104x over previous
import jax
import jax.numpy as jnp
from jax.experimental import pallas as pl
from jax.experimental.pallas import tpu as pltpu
from jax.experimental.pallas import tpu_sc as plsc

F32 = jnp.float32
BF16 = jnp.bfloat16

_B, _T, _D, _DFF, _H, _E, _K = 2, 2048, 768, 3072, 12, 8, 2
_DH = _D // _H
_N = _B * _T
_NP = _N * _K
_BM = 512
_NBLK = _NP // _BM + _E
_NPAD = _NBLK * _BM
_BQ = 1024
_DH2 = _D // 2


def _qkv_body(x_ref, w_ref, b_ref, o_ref):
    acc = jax.lax.dot_general(x_ref[...].astype(BF16), w_ref[...],
                              (((1,), (0,)), ((), ())),
                              preferred_element_type=F32)
    o_ref[...] = (acc + b_ref[...]).astype(BF16)


def _qkv_proj(xb, wqkv, bqkv):
    bm = 512
    return pl.pallas_call(
        _qkv_body,
        grid=(_N // bm,),
        in_specs=[pl.BlockSpec((bm, _D), lambda i: (i, 0)),
                  pl.BlockSpec((_D, 3 * _D), lambda i: (0, 0)),
                  pl.BlockSpec((1, 3 * _D), lambda i: (0, 0))],
        out_specs=pl.BlockSpec((bm, 3 * _D), lambda i: (i, 0)),
        out_shape=jax.ShapeDtypeStruct((_N, 3 * _D), BF16),
    )(xb, wqkv, bqkv)


def _attn_body(q_ref, k_ref, v_ref, a_ref, c_ref):
    qq = q_ref[...]
    kk = k_ref[...]
    vv = v_ref[...]
    outs = []
    for hh in range(2):
        sl = slice(hh * _DH, (hh + 1) * _DH)
        s = jax.lax.dot_general(qq[:, sl], kk[:, sl], (((1,), (1,)), ((), ())),
                                preferred_element_type=F32)
        p = jnp.exp2(s * 0.18033688011112042)
        p = p * (1.0 / jnp.sum(p, axis=-1, keepdims=True))
        a_ref[0, hh] = p
        outs.append(jax.lax.dot_general(p.astype(BF16), vv[:, sl],
                                        (((1,), (0,)), ((), ())),
                                        preferred_element_type=F32))
    c_ref[...] = jnp.concatenate(outs, axis=1).astype(BF16)


def _attention(qkv):
    nj = _T // _BQ
    hp = _H // 2
    return pl.pallas_call(
        _attn_body,
        grid=(_B, hp, nj),
        in_specs=[
            pl.BlockSpec((_BQ, 2 * _DH), lambda b, h, j, _nj=nj: (b * _nj + j, h)),
            pl.BlockSpec((_T, 2 * _DH), lambda b, h, j, _hp=hp: (b, _hp + h)),
            pl.BlockSpec((_T, 2 * _DH), lambda b, h, j, _hp=hp: (b, 2 * _hp + h)),
        ],
        out_specs=[
            pl.BlockSpec((1, 2, _BQ, _T), lambda b, h, j: (b, h, j, 0)),
            pl.BlockSpec((_BQ, 2 * _DH), lambda b, h, j, _nj=nj: (b * _nj + j, h)),
        ],
        out_shape=[
            jax.ShapeDtypeStruct((_B, _H, _T, _T), F32),
            jax.ShapeDtypeStruct((_N, _D), BF16),
        ],
    )(qkv, qkv, qkv)


def _post_body(ctx_ref, x_ref, wo_ref, bo_ref, g1_ref, be1_ref, wg_ref,
               x12_ref, gate_ref):
    nx = jax.lax.dot_general(ctx_ref[...], wo_ref[...], (((1,), (0,)), ((), ())),
                             preferred_element_type=F32) + bo_ref[...]
    x1 = x_ref[...] + nx
    mu = jnp.mean(x1, axis=-1, keepdims=True)
    xc = x1 - mu
    var = jnp.mean(xc * xc, axis=-1, keepdims=True)
    x1 = xc / jnp.sqrt(var + 1e-5) * g1_ref[...] + be1_ref[...]
    x12_ref[0, 0] = x1[:, :_DH2]
    x12_ref[1, 0] = x1[:, _DH2:]
    logits = jax.lax.dot_general(x1, wg_ref[...], (((1,), (0,)), ((), ())),
                                 preferred_element_type=F32)
    mask = jax.lax.broadcasted_iota(jnp.int32, logits.shape, 1) < _E
    logits = jnp.where(mask, logits, -1e30)
    mx = jnp.max(logits, axis=-1, keepdims=True)
    p = jnp.exp(logits - mx)
    gate_ref[...] = p / jnp.sum(p, axis=-1, keepdims=True)


def _post_attn(ctx2, xf, wob, bo2, g12, be12, wgp):
    bm = 512
    return pl.pallas_call(
        _post_body,
        grid=(_N // bm,),
        in_specs=[
            pl.BlockSpec((bm, _D), lambda i: (i, 0)),
            pl.BlockSpec((bm, _D), lambda i: (i, 0)),
            pl.BlockSpec((_D, _D), lambda i: (0, 0)),
            pl.BlockSpec((1, _D), lambda i: (0, 0)),
            pl.BlockSpec((1, _D), lambda i: (0, 0)),
            pl.BlockSpec((1, _D), lambda i: (0, 0)),
            pl.BlockSpec((_D, 128), lambda i: (0, 0)),
        ],
        out_specs=[
            pl.BlockSpec((2, 1, bm, _DH2), lambda i: (0, i, 0, 0)),
            pl.BlockSpec((bm, 128), lambda i: (i, 0)),
        ],
        out_shape=[
            jax.ShapeDtypeStruct((2, _N // bm, bm, _DH2), F32),
            jax.ShapeDtypeStruct((_N, 128), F32),
        ],
    )(ctx2, xf, wob, bo2, g12, be12, wgp)


def _sc_gather(table, idx, n_rows, window=128):
    mesh = plsc.VectorSubcoreMesh(core_axis_name="core",
                                  subcore_axis_name="subcore")
    width = table.shape[1]
    idx2 = idx.reshape(1, n_rows)

    @pl.kernel(out_type=jax.ShapeDtypeStruct((n_rows, width), table.dtype),
               mesh=mesh)
    def k(x_hbm, i_hbm, o_hbm):
        def body(i_vmem, o_vmem):
            pltpu.sync_copy(x_hbm.at[i_vmem.at[0]], o_vmem)

        pltpu.emit_pipeline(
            body,
            grid=(n_rows // window,),
            in_specs=[pl.BlockSpec((1, window), lambda i: (0, i))],
            out_specs=[pl.BlockSpec((window, width), lambda i: (i, 0))],
            core_axis_name=("core", "subcore"),
            dimension_semantics=(pltpu.PARALLEL,),
        )(i_hbm, o_hbm)

    return k(table, idx2)


def _sc_scatter(x2, ia, ib, n_slots, window=128):
    mesh = plsc.VectorSubcoreMesh(core_axis_name="core",
                                  subcore_axis_name="subcore")
    n_rows, width = x2.shape
    ia2 = ia.reshape(1, n_rows)
    ib2 = ib.reshape(1, n_rows)

    @pl.kernel(out_type=jax.ShapeDtypeStruct((n_slots, width), x2.dtype),
               mesh=mesh)
    def k(x_hbm, ia_hbm, ib_hbm, o_hbm):
        def body(x_vmem, ia_vmem, ib_vmem):
            pltpu.sync_copy(x_vmem, o_hbm.at[ia_vmem.at[0]])
            pltpu.sync_copy(x_vmem, o_hbm.at[ib_vmem.at[0]])

        pltpu.emit_pipeline(
            body,
            grid=(n_rows // window,),
            in_specs=[
                pl.BlockSpec((window, width), lambda i: (i, 0)),
                pl.BlockSpec((1, window), lambda i: (0, i)),
                pl.BlockSpec((1, window), lambda i: (0, i)),
            ],
            out_specs=[],
            core_axis_name=("core", "subcore"),
            dimension_semantics=(pltpu.PARALLEL,),
        )(x_hbm, ia_hbm, ib_hbm)

    return k(x2, ia2, ib2)


def _ffn_body(be_ref, bn_ref, lo_ref, hi_ref, w1lo_ref, w1hi_ref,
              b1_ref, w2_ref, b2_ref, ov_ref, w1lo_bf, w1hi_bf, w2_bf):
    i = pl.program_id(0)
    prev = jnp.maximum(i - 1, 0)
    changed = (i == 0) | (be_ref[i] != be_ref[prev])

    @pl.when(changed)
    def _():
        w1lo_bf[...] = w1lo_ref[0, 0].astype(BF16)
        w1hi_bf[...] = w1hi_ref[0, 0].astype(BF16)
        w2_bf[...] = w2_ref[0].astype(BF16)

    @pl.when(bn_ref[i] > 0)
    def _():
        h = jax.lax.dot_general(lo_ref[...].astype(BF16), w1lo_bf[...],
                                (((1,), (0,)), ((), ())),
                                preferred_element_type=F32)
        h += jax.lax.dot_general(hi_ref[...].astype(BF16), w1hi_bf[...],
                                 (((1,), (0,)), ((), ())),
                                 preferred_element_type=F32)
        h = jnp.maximum(h + b1_ref[0], 0.0).astype(BF16)
        o = jax.lax.dot_general(h, w2_bf[...], (((1,), (0,)), ((), ())),
                                preferred_element_type=F32)
        o = o + b2_ref[0]
        ov_ref[0, 0] = o[:, :_DH2]
        ov_ref[1, 0] = o[:, _DH2:]


def _ffn(block_expert, block_nrows, rows2, w1b, b1r, w2b, b2r):
    grid_spec = pltpu.PrefetchScalarGridSpec(
        num_scalar_prefetch=2,
        grid=(_NBLK,),
        in_specs=[
            pl.BlockSpec((_BM, _DH2), lambda i, be, bn: (i, 0)),
            pl.BlockSpec((_BM, _DH2), lambda i, be, bn: (i + _NBLK, 0)),
            pl.BlockSpec((1, 1, _DH2, _DFF), lambda i, be, bn: (be[i], 0, 0, 0)),
            pl.BlockSpec((1, 1, _DH2, _DFF), lambda i, be, bn: (be[i], 1, 0, 0)),
            pl.BlockSpec((1, 1, _DFF), lambda i, be, bn: (be[i], 0, 0)),
            pl.BlockSpec((1, _DFF, _D), lambda i, be, bn: (be[i], 0, 0)),
            pl.BlockSpec((1, 1, _D), lambda i, be, bn: (be[i], 0, 0)),
        ],
        out_specs=pl.BlockSpec((2, 1, _BM, _DH2), lambda i, be, bn: (0, i, 0, 0)),
        scratch_shapes=[
            pltpu.VMEM((_DH2, _DFF), BF16),
            pltpu.VMEM((_DH2, _DFF), BF16),
            pltpu.VMEM((_DFF, _D), BF16),
        ],
    )
    return pl.pallas_call(
        _ffn_body,
        grid_spec=grid_spec,
        out_shape=jax.ShapeDtypeStruct((2, _NBLK, _BM, _DH2), F32),
    )(block_expert, block_nrows, rows2, rows2, w1b, w1b, b1r, w2b, b2r)


def _final_body(xlo_ref, xhi_ref, galo_ref, gahi_ref, gblo_ref, gbhi_ref,
                wa_ref, wb_ref, g2_ref, be2_ref, o_ref):
    wa = wa_ref[:, 0:1]
    wb = wb_ref[:, 0:1]
    slo = xlo_ref[...] + wa * galo_ref[...] + wb * gblo_ref[...]
    shi = xhi_ref[...] + wa * gahi_ref[...] + wb * gbhi_ref[...]
    mu = (jnp.sum(slo, axis=-1, keepdims=True)
          + jnp.sum(shi, axis=-1, keepdims=True)) * (1.0 / _D)
    clo = slo - mu
    chi = shi - mu
    var = (jnp.sum(clo * clo, axis=-1, keepdims=True)
           + jnp.sum(chi * chi, axis=-1, keepdims=True)) * (1.0 / _D)
    r = 1.0 / jnp.sqrt(var + 1e-5)
    o_ref[:, :_DH2] = clo * r * g2_ref[:, :_DH2] + be2_ref[:, :_DH2]
    o_ref[:, _DH2:] = chi * r * g2_ref[:, _DH2:] + be2_ref[:, _DH2:]


def _final(x12f, g4, wab, g22, be22):
    bm = 512
    nb = _N // bm
    return pl.pallas_call(
        _final_body,
        grid=(nb,),
        in_specs=[
            pl.BlockSpec((bm, _DH2), lambda i: (i, 0)),
            pl.BlockSpec((bm, _DH2), lambda i, _nb=nb: (i + _nb, 0)),
            pl.BlockSpec((bm, _DH2), lambda i: (i, 0)),
            pl.BlockSpec((bm, _DH2), lambda i, _nb=nb: (i + 2 * _nb, 0)),
            pl.BlockSpec((bm, _DH2), lambda i, _nb=nb: (i + _nb, 0)),
            pl.BlockSpec((bm, _DH2), lambda i, _nb=nb: (i + 3 * _nb, 0)),
            pl.BlockSpec((bm, 128), lambda i: (i, 0)),
            pl.BlockSpec((bm, 128), lambda i, _nb=nb: (i + _nb, 0)),
            pl.BlockSpec((1, _D), lambda i: (0, 0)),
            pl.BlockSpec((1, _D), lambda i: (0, 0)),
        ],
        out_specs=pl.BlockSpec((bm, _D), lambda i: (i, 0)),
        out_shape=jax.ShapeDtypeStruct((_N, _D), F32),
    )(x12f, x12f, g4, g4, g4, g4, wab, wab, g22, be22)


def _routing(probs):
    w, sel = jax.lax.top_k(probs, _K)
    e_flat = sel.reshape(_NP).astype(jnp.int32)
    oh = (e_flat[:, None] == jnp.arange(_E, dtype=jnp.int32)[None, :])
    oh = oh.astype(F32)
    ch = 128
    nch = _NP // ch
    oh3 = oh.reshape(nch, ch, _E)
    ii = jax.lax.broadcasted_iota(jnp.int32, (ch, ch), 0)
    jj = jax.lax.broadcasted_iota(jnp.int32, (ch, ch), 1)
    tril = (ii >= jj).astype(F32)
    within = jnp.einsum("ij,bjk->bik", tril, oh3,
                        preferred_element_type=F32)
    chunk_tot = jnp.sum(oh3, axis=1)
    i2 = jax.lax.broadcasted_iota(jnp.int32, (nch, nch), 0)
    j2 = jax.lax.broadcasted_iota(jnp.int32, (nch, nch), 1)
    stril = (i2 > j2).astype(F32)
    offs = jnp.einsum("ij,jk->ik", stril, chunk_tot,
                      preferred_element_type=F32)
    ranks = (within + offs[:, None, :]).reshape(_NP, _E)
    c = (jnp.sum(chunk_tot, axis=0)).astype(jnp.int32)
    blocks = (c + _BM - 1) // _BM
    cum_blocks = jnp.cumsum(blocks)
    pad_start = _BM * (cum_blocks - blocks)
    rank_p = jnp.sum(ranks * oh, axis=1).astype(jnp.int32) - 1
    dest = pad_start[e_flat] + rank_p
    bidx = jnp.arange(_NBLK, dtype=jnp.int32)
    block_expert = jnp.minimum(
        jnp.searchsorted(cum_blocks, bidx, side="right"), _E - 1
    ).astype(jnp.int32)
    off = bidx * _BM - pad_start[block_expert]
    block_nrows = jnp.clip(c[block_expert] - off, 0, _BM).astype(jnp.int32)
    return w, dest, block_expert, block_nrows


def kernel(x, Wq, bq, Wk, bk, Wv, bv, Wo, bo, g1, be1, g2, be2, Wg, W1, b1,
           W2, b2):
    xf = x.reshape(_N, _D)
    wqkv = jnp.concatenate([Wq, Wk, Wv], axis=1).astype(BF16)
    bqkv = jnp.concatenate([bq, bk, bv])[None, :]
    qkv = _qkv_proj(xf, wqkv, bqkv)
    attn, ctx2 = _attention(qkv)

    wgp = jnp.pad(Wg, ((0, 0), (0, 128 - _E)))
    x12, gate = _post_attn(ctx2, xf, Wo.astype(BF16), bo[None, :],
                           g1[None, :], be1[None, :], wgp)
    x12f = x12.reshape(2 * _N, _DH2)
    probs = gate[:, :_E]

    w, dest, block_expert, block_nrows = _routing(probs)

    d2 = dest.reshape(_N, _K)
    ia = jnp.concatenate([d2[:, 0], d2[:, 0] + _NPAD])
    ib = jnp.concatenate([d2[:, 1], d2[:, 1] + _NPAD])
    rows2 = _sc_scatter(x12f, ia, ib, 2 * _NPAD)
    ov = _ffn(block_expert, block_nrows, rows2,
              W1.reshape(_E, 2, _DH2, _DFF), b1.reshape(_E, 1, _DFF),
              W2, b2.reshape(_E, 1, _D))
    ovf = ov.reshape(2 * _NPAD, _DH2)

    idx2 = d2.T.reshape(_NP)
    gidx = jnp.concatenate([idx2, idx2 + _NPAD])
    g4 = _sc_gather(ovf, gidx, 2 * _NP)

    wab = jnp.broadcast_to(w.T.reshape(_NP, 1), (_NP, 128))
    out = _final(x12f, g4, wab, g2[None, :], be2[None, :])
    return (out.reshape(_B, _T, _D), attn, probs.reshape(_B, _T, _E))
